# Initial kernel scaffold; baseline (speedup 1.0000x reference)
import jax
import jax.numpy as jnp
from jax.experimental import pallas as pl


def kernel(atomic_num, formal_charge, degree, chiral_tag, total_numHs, is_aromatic, hybridization, W_atomic_num, W_formal_charge, W_degree, W_chiral_tag, W_total_numHs, W_is_aromatic, W_hybridization):
    raise NotImplementedError("write your pallas kernel here")


# speedup vs baseline: 1.3472x; 1.3472x over previous
import functools

import jax
import jax.numpy as jnp
from jax import lax
from jax.experimental import pallas as pl
from jax.experimental.pallas import tpu as pltpu
from jax.experimental.pallas import tpu_sc as plsc

N = 100000
D = 128
C = 80
NUM_CHUNKS = N // C
NC, NS, L = 2, 16, 16
NW = NC * NS
F = 7


def _body(a0, a1, a2, a3, a4, a5, a6,
          w0, w1, w2, w3, w4, w5, w6,
          out_hbm, idx_v, rows_v, sem):
    idx_hbm = (a0, a1, a2, a3, a4, a5, a6)
    tables = (w0, w1, w2, w3, w4, w5, w6)
    wid = lax.axis_index("s") * NC + lax.axis_index("c")
    nchunks = (NUM_CHUNKS - wid + NW - 1) // NW

    def chunk_body(i, _):
        c = wid + i * NW
        base = c * C
        for f in range(F):
            pltpu.async_copy(idx_hbm[f].at[pl.ds(base, C)], idx_v.at[f], sem)
        for f in range(F):
            pltpu.make_async_copy(idx_hbm[f].at[pl.ds(base, C)],
                                  idx_v.at[f], sem).wait()
        for f in range(F):
            pltpu.async_copy(tables[f].at[idx_v.at[f]], rows_v.at[f], sem)
        for f in range(F):
            pltpu.make_async_copy(tables[f].at[idx_v.at[f]],
                                  rows_v.at[f], sem).wait()

        def sum_body(j, _):
            r = j // (D // L)
            k = (j % (D // L)) * L
            acc = rows_v[0, r, pl.ds(k, L)]
            for f in range(1, F):
                acc = acc + rows_v[f, r, pl.ds(k, L)]
            rows_v[0, r, pl.ds(k, L)] = acc
            return 0

        lax.fori_loop(0, C * D // L, sum_body, 0)
        pltpu.sync_copy(rows_v.at[0], out_hbm.at[pl.ds(base, C)])
        return 0

    lax.fori_loop(0, nchunks, chunk_body, 0)


@jax.jit
def kernel(atomic_num, formal_charge, degree, chiral_tag, total_numHs,
           is_aromatic, hybridization,
           W_atomic_num, W_formal_charge, W_degree, W_chiral_tag,
           W_total_numHs, W_is_aromatic, W_hybridization):
    mesh = plsc.VectorSubcoreMesh(core_axis_name="c", subcore_axis_name="s")
    run = pl.kernel(
        _body,
        out_type=jax.ShapeDtypeStruct((N, D), jnp.float32),
        mesh=mesh,
        scratch_types=[
            pltpu.VMEM((F, C), jnp.int32),
            pltpu.VMEM((F, C, D), jnp.float32),
            pltpu.SemaphoreType.DMA,
        ],
    )
    return run(atomic_num, formal_charge, degree, chiral_tag, total_numHs,
               is_aromatic, hybridization,
               W_atomic_num, W_formal_charge, W_degree, W_chiral_tag,
               W_total_numHs, W_is_aromatic, W_hybridization)


# speedup vs baseline: 7.0493x; 5.2327x over previous
import functools

import jax
import jax.numpy as jnp
from jax import lax
from jax.experimental import pallas as pl
from jax.experimental.pallas import tpu as pltpu
from jax.experimental.pallas import tpu_sc as plsc

N = 100000
D = 128
C = 80
NUM_CHUNKS = N // C
NC, NS, L = 2, 16, 16
NW = NC * NS
F = 7
G = 3


def _body(a0, a1, a2, a3, a4, a5, a6,
          t0, t1, t2,
          out_hbm, idx_v, fidx_v, rows_v, sem):
    idx_hbm = (a0, a1, a2, a3, a4, a5, a6)
    tables = (t0, t1, t2)
    wid = lax.axis_index("s") * NC + lax.axis_index("c")
    nchunks = (NUM_CHUNKS - wid + NW - 1) // NW

    def chunk_body(i, _):
        c = wid + i * NW
        base = c * C
        for f in range(F):
            pltpu.async_copy(idx_hbm[f].at[pl.ds(base, C)], idx_v.at[f], sem)
        for f in range(F):
            pltpu.make_async_copy(idx_hbm[f].at[pl.ds(base, C)],
                                  idx_v.at[f], sem).wait()

        for s in range(C // L):
            sl = pl.ds(s * L, L)
            fidx_v[0, sl] = idx_v[0, sl]
            fidx_v[1, sl] = (idx_v[1, sl] * 17 + idx_v[2, sl]) * 14 + idx_v[3, sl]
            fidx_v[2, sl] = (idx_v[4, sl] * 7 + idx_v[5, sl]) * 14 + idx_v[6, sl]

        for g in range(G):
            pltpu.async_copy(tables[g].at[fidx_v.at[g]], rows_v.at[g], sem)
        for g in range(G):
            pltpu.make_async_copy(tables[g].at[fidx_v.at[g]],
                                  rows_v.at[g], sem).wait()

        def sum_body(r, _):
            for s in range(D // L):
                sl = pl.ds(s * L, L)
                rows_v[0, r, sl] = (rows_v[0, r, sl] + rows_v[1, r, sl]
                                    + rows_v[2, r, sl])
            return 0

        lax.fori_loop(0, C, sum_body, 0)
        pltpu.sync_copy(rows_v.at[0], out_hbm.at[pl.ds(base, C)])
        return 0

    lax.fori_loop(0, nchunks, chunk_body, 0)


@jax.jit
def kernel(atomic_num, formal_charge, degree, chiral_tag, total_numHs,
           is_aromatic, hybridization,
           W_atomic_num, W_formal_charge, W_degree, W_chiral_tag,
           W_total_numHs, W_is_aromatic, W_hybridization):
    t1 = (W_formal_charge[:, None, None, :] + W_degree[None, :, None, :]
          + W_chiral_tag[None, None, :, :]).reshape(-1, D)
    t2 = (W_total_numHs[:, None, None, :] + W_is_aromatic[None, :, None, :]
          + W_hybridization[None, None, :, :]).reshape(-1, D)

    mesh = plsc.VectorSubcoreMesh(core_axis_name="c", subcore_axis_name="s")
    run = pl.kernel(
        _body,
        out_type=jax.ShapeDtypeStruct((N, D), jnp.float32),
        mesh=mesh,
        scratch_types=[
            pltpu.VMEM((F, C), jnp.int32),
            pltpu.VMEM((G, C), jnp.int32),
            pltpu.VMEM((G, C, D), jnp.float32),
            pltpu.SemaphoreType.DMA,
        ],
    )
    return run(atomic_num, formal_charge, degree, chiral_tag, total_numHs,
               is_aromatic, hybridization, W_atomic_num, t1, t2)
